# baseline (speedup 1.0000x reference)
import jax
import jax.numpy as jnp
from jax import lax
from jax.experimental import pallas as pl
from jax.experimental.pallas import tpu as pltpu
from jax.experimental.pallas import tpu_sc as plsc

PC_RANGE = [-51.2, -51.2, -5.0, 51.2, 51.2, 3.0]
IMG_H, IMG_W = 256, 704
EPS = 1e-06

N_CAM = 6
HF, WF = 32, 88
HW = HF * WF
C = 256
R_TOT = 64 * 64 * 4
TILE_R = 256
TAPS = 4 * N_CAM

NW = 32
QCH = 64
NCHT = R_TOT // QCH
NCH = NCHT // NW
TPC = QCH * TAPS
GB = 64


def _tc_kernel(rp_ref, pts4_ref, l2it_ref, w1_ref, b1_ref, w2_ref, b2_ref,
               pos_ref, uv_ref, tw_ref, ti_ref):
    rp = rp_ref[...]
    mid = jnp.maximum(
        jnp.dot(rp, w1_ref[...], preferred_element_type=jnp.float32)
        + b1_ref[...][None, :], 0.0)
    pos_ref[...] = (jnp.dot(mid, w2_ref[...],
                            preferred_element_type=jnp.float32)
                    + b2_ref[...][None, :])

    pts4 = pts4_ref[...]

    for n in range(N_CAM):
        pc = jnp.dot(pts4, l2it_ref[n], preferred_element_type=jnp.float32)
        pc0 = pc[:, 0:1]
        pc1 = pc[:, 1:2]
        depth = pc[:, 2:3]
        maxd = jnp.maximum(depth, EPS)
        u = (pc0 / maxd) / IMG_W
        v = (pc1 / maxd) / IMG_H
        uv_ref[:, 2 * n:2 * n + 1] = u
        uv_ref[:, 2 * n + 1:2 * n + 2] = v
        maskf = ((depth > EPS) & (u >= 0.0) & (u <= 1.0)
                 & (v >= 0.0) & (v <= 1.0)).astype(jnp.float32)
        x = u * WF - 0.5
        y = v * HF - 0.5
        x0 = jnp.floor(x)
        y0 = jnp.floor(y)
        wx1 = x - x0
        wx0 = 1.0 - wx1
        wy1 = y - y0
        wy0 = 1.0 - wy1
        inx0 = ((x0 >= 0) & (x0 <= WF - 1)).astype(jnp.float32) * maskf
        inx1 = ((x0 + 1.0 >= 0) & (x0 + 1.0 <= WF - 1)).astype(jnp.float32)
        iny0 = ((y0 >= 0) & (y0 <= HF - 1)).astype(jnp.float32)
        iny1 = ((y0 + 1.0 >= 0) & (y0 + 1.0 <= HF - 1)).astype(jnp.float32)
        x0i = jnp.clip(x0, 0, WF - 1).astype(jnp.int32)
        x1i = jnp.clip(x0 + 1.0, 0, WF - 1).astype(jnp.int32)
        y0i = jnp.clip(y0, 0, HF - 1).astype(jnp.int32)
        y1i = jnp.clip(y0 + 1.0, 0, HF - 1).astype(jnp.int32)
        base = n * HW
        j = 4 * n
        tw_ref[:, j:j + 1] = wy0 * wx0 * iny0 * inx0
        tw_ref[:, j + 1:j + 2] = wy0 * wx1 * iny0 * inx1 * maskf
        tw_ref[:, j + 2:j + 3] = wy1 * wx0 * iny1 * inx0
        tw_ref[:, j + 3:j + 4] = wy1 * wx1 * iny1 * inx1 * maskf
        ti_ref[:, j:j + 1] = base + y0i * WF + x0i
        ti_ref[:, j + 1:j + 2] = base + y0i * WF + x1i
        ti_ref[:, j + 2:j + 3] = base + y1i * WF + x0i
        ti_ref[:, j + 3:j + 4] = base + y1i * WF + x1i


def _tc_stage(rp_flat, pts4, l2it, W1, b1, W2, b2):
    grid = (R_TOT // TILE_R,)
    return pl.pallas_call(
        _tc_kernel,
        grid=grid,
        in_specs=[
            pl.BlockSpec((TILE_R, 3), lambda i: (i, 0)),
            pl.BlockSpec((TILE_R, 4), lambda i: (i, 0)),
            pl.BlockSpec((N_CAM, 4, 4), lambda i: (0, 0, 0)),
            pl.BlockSpec((3, 512), lambda i: (0, 0)),
            pl.BlockSpec((512,), lambda i: (0,)),
            pl.BlockSpec((512, C), lambda i: (0, 0)),
            pl.BlockSpec((C,), lambda i: (0,)),
        ],
        out_specs=[
            pl.BlockSpec((TILE_R, C), lambda i: (i, 0)),
            pl.BlockSpec((TILE_R, 2 * N_CAM), lambda i: (i, 0)),
            pl.BlockSpec((TILE_R, TAPS), lambda i: (i, 0)),
            pl.BlockSpec((TILE_R, TAPS), lambda i: (i, 0)),
        ],
        out_shape=[
            jax.ShapeDtypeStruct((R_TOT, C), jnp.float32),
            jax.ShapeDtypeStruct((R_TOT, 2 * N_CAM), jnp.float32),
            jax.ShapeDtypeStruct((R_TOT, TAPS), jnp.float32),
            jax.ShapeDtypeStruct((R_TOT, TAPS), jnp.int32),
        ],
    )(rp_flat, pts4, l2it, W1, b1, W2, b2)


def _sc_body(fm_hbm, ci_hbm, wb_hbm, qb_hbm, nb_hbm, zeros_hbm, out_hbm,
             civ, wv, qv, rows_v, acc, nbv, sem):
    wid = lax.axis_index("s") * 2 + lax.axis_index("c")
    pltpu.sync_copy(nb_hbm, nbv)

    def chunk_body(k, _):
        ch = wid * NCH + k
        qbase = ch * QCH
        pltpu.sync_copy(ci_hbm.at[ch], civ)
        pltpu.sync_copy(wb_hbm.at[ch], wv)
        pltpu.sync_copy(qb_hbm.at[ch], qv)
        pltpu.sync_copy(zeros_hbm, acc)
        nb = plsc.load_gather(nbv, [jnp.full((16,), ch, jnp.int32)])[0]

        def batch(b, _):
            tstart = b * GB
            pltpu.async_copy(fm_hbm.at[civ.at[pl.ds(tstart, GB)]],
                             rows_v, sem).wait()

            def tap(t, _):
                base = tstart + t
                w16 = wv[pl.ds(base * 16, 16)]
                qb = qv[pl.ds(base * 16, 16)]
                for c16 in range(C // 16):
                    row = rows_v[t, pl.ds(c16 * 16, 16)]
                    plsc.addupdate_scatter(acc, [qb + c16 * 16], w16 * row)
                return 0

            lax.fori_loop(0, GB, tap, 0, unroll=False)
            return 0

        lax.fori_loop(0, nb, batch, 0, unroll=False)
        pltpu.sync_copy(acc, out_hbm.at[pl.ds(qbase * C, QCH * C)])
        return 0

    lax.fori_loop(0, NCH, chunk_body, 0, unroll=False)


def _sc_stage(fm_flat, ci, wb, qb, nb, zeros_hbm):
    smesh = plsc.VectorSubcoreMesh(core_axis_name="c", subcore_axis_name="s")
    run = pl.kernel(
        _sc_body,
        mesh=smesh,
        compiler_params=pltpu.CompilerParams(needs_layout_passes=False),
        out_type=jax.ShapeDtypeStruct((R_TOT * C,), jnp.float32),
        scratch_types=[
            pltpu.VMEM((TPC,), jnp.int32),
            pltpu.VMEM((TPC * 16,), jnp.float32),
            pltpu.VMEM((TPC * 16,), jnp.int32),
            pltpu.VMEM((GB, C), jnp.float32),
            pltpu.VMEM((QCH * C,), jnp.float32),
            pltpu.VMEM((NCHT,), jnp.int32),
            pltpu.SemaphoreType.DMA,
        ],
    )
    return run(fm_flat, ci, wb, qb, nb, zeros_hbm)


def _compact(tw, ti):
    w = tw.reshape(NCHT, TPC)
    ii = ti.reshape(NCHT, TPC)
    qmod = (jnp.arange(R_TOT, dtype=jnp.int32) % QCH)
    qq = jnp.broadcast_to(qmod[:, None], (R_TOT, TAPS)).reshape(NCHT, TPC)
    valid = w != 0.0
    pos = jnp.cumsum(valid, axis=1, dtype=jnp.int32) - valid
    dest = jnp.where(valid, pos, TPC)
    rows = jnp.broadcast_to(jnp.arange(NCHT, dtype=jnp.int32)[:, None],
                            (NCHT, TPC))
    ci = jnp.zeros((NCHT, TPC), jnp.int32).at[rows, dest].set(
        ii, mode="drop", unique_indices=True)
    wc = jnp.zeros((NCHT, TPC), jnp.float32).at[rows, dest].set(
        w, mode="drop", unique_indices=True)
    qc = jnp.zeros((NCHT, TPC), jnp.int32).at[rows, dest].set(
        qq, mode="drop", unique_indices=True)
    counts = jnp.sum(valid, axis=1, dtype=jnp.int32)
    nb = (counts + GB - 1) // GB
    iota16 = jnp.arange(16, dtype=jnp.int32)
    wb = jnp.broadcast_to(wc[:, :, None], (NCHT, TPC, 16)).reshape(
        NCHT, TPC * 16)
    qb = (qc[:, :, None] * C + iota16[None, None, :]).reshape(NCHT, TPC * 16)
    return ci, wb, qb, nb


@jax.jit
def kernel(mlvl_feats, reference_points, lidar2img, W1, b1, W2, b2):
    rp_flat = reference_points[0].transpose(1, 2, 0, 3).reshape(R_TOT, 3)
    pts4 = jnp.concatenate([
        rp_flat[:, 0:1] * (PC_RANGE[3] - PC_RANGE[0]) + PC_RANGE[0],
        rp_flat[:, 1:2] * (PC_RANGE[4] - PC_RANGE[1]) + PC_RANGE[1],
        rp_flat[:, 2:3] * (PC_RANGE[5] - PC_RANGE[2]) + PC_RANGE[2],
        jnp.ones((R_TOT, 1), jnp.float32),
    ], axis=-1)
    l2it = jnp.transpose(lidar2img[0], (0, 2, 1))
    fm_flat = mlvl_feats[0].transpose(0, 2, 3, 1).reshape(N_CAM * HW, C)

    pos, uv, tw, ti = _tc_stage(rp_flat, pts4, l2it, W1, b1, W2, b2)
    ci, wb, qb, nb = _compact(tw, ti)
    zeros_hbm = jnp.zeros((QCH * C,), jnp.float32)
    sampled = _sc_stage(fm_flat, ci, wb, qb, nb, zeros_hbm)

    sf = (sampled.reshape(R_TOT, C) + pos).reshape(1, 64, 64, 4, C)
    sf = sf.transpose(0, 4, 3, 1, 2)
    spc = uv.reshape(4096, 4, N_CAM, 2).transpose(2, 0, 1, 3)
    spc = spc.reshape(1, N_CAM, 4096, 1, 4, 2)
    return sf, spc


# speedup vs baseline: 8.7162x; 8.7162x over previous
import jax
import jax.numpy as jnp
from jax.experimental import pallas as pl

PC_RANGE = [-51.2, -51.2, -5.0, 51.2, 51.2, 3.0]
IMG_H, IMG_W = 256, 704
EPS = 1e-06

N_CAM = 6
HF, WF = 32, 88
HW = HF * WF
C = 256
R_TOT = 64 * 64 * 4
TILE_R = 256


def _bev_kernel(rp_ref, pts4_ref, l2it_ref, fm_ref, w1_ref, b1_ref, w2_ref,
                b2_ref, out_ref, uv_ref):
    rp = rp_ref[...]
    mid = jnp.maximum(
        jnp.dot(rp, w1_ref[...], preferred_element_type=jnp.float32)
        + b1_ref[...][None, :], 0.0)
    acc = (jnp.dot(mid, w2_ref[...], preferred_element_type=jnp.float32)
           + b2_ref[...][None, :])

    pts4 = pts4_ref[...]

    coli = jax.lax.broadcasted_iota(jnp.int32, (1, HW), 1)
    hci = coli // WF
    hc = hci.astype(jnp.float32)
    xc = (coli - hci * WF).astype(jnp.float32)

    for n in range(N_CAM):
        pc = jnp.dot(pts4, l2it_ref[n], preferred_element_type=jnp.float32)
        pc0 = pc[:, 0:1]
        pc1 = pc[:, 1:2]
        depth = pc[:, 2:3]
        maxd = jnp.maximum(depth, EPS)
        u = (pc0 / maxd) / IMG_W
        v = (pc1 / maxd) / IMG_H
        uv_ref[:, 2 * n:2 * n + 1] = u
        uv_ref[:, 2 * n + 1:2 * n + 2] = v
        mask = ((depth > EPS) & (u >= 0.0) & (u <= 1.0)
                & (v >= 0.0) & (v <= 1.0))
        x = u * WF - 0.5
        y = v * HF - 0.5
        y = jnp.where(mask, y, -1e9)
        a = (jnp.maximum(1.0 - jnp.abs(y - hc), 0.0)
             * jnp.maximum(1.0 - jnp.abs(x - xc), 0.0))
        acc = acc + jnp.dot(a.astype(jnp.bfloat16), fm_ref[n],
                            preferred_element_type=jnp.float32)

    out_ref[...] = acc


@jax.jit
def kernel(mlvl_feats, reference_points, lidar2img, W1, b1, W2, b2):
    rp_flat = reference_points[0].transpose(1, 2, 0, 3).reshape(R_TOT, 3)
    pts4 = jnp.concatenate([
        rp_flat[:, 0:1] * (PC_RANGE[3] - PC_RANGE[0]) + PC_RANGE[0],
        rp_flat[:, 1:2] * (PC_RANGE[4] - PC_RANGE[1]) + PC_RANGE[1],
        rp_flat[:, 2:3] * (PC_RANGE[5] - PC_RANGE[2]) + PC_RANGE[2],
        jnp.ones((R_TOT, 1), jnp.float32),
    ], axis=-1)
    fm = mlvl_feats[0].transpose(0, 2, 3, 1).reshape(N_CAM, HW, C)
    fm = fm.astype(jnp.bfloat16)
    l2it = jnp.transpose(lidar2img[0], (0, 2, 1))

    grid = (R_TOT // TILE_R,)
    out, uv = pl.pallas_call(
        _bev_kernel,
        grid=grid,
        in_specs=[
            pl.BlockSpec((TILE_R, 3), lambda i: (i, 0)),
            pl.BlockSpec((TILE_R, 4), lambda i: (i, 0)),
            pl.BlockSpec((N_CAM, 4, 4), lambda i: (0, 0, 0)),
            pl.BlockSpec((N_CAM, HW, C), lambda i: (0, 0, 0)),
            pl.BlockSpec((3, 512), lambda i: (0, 0)),
            pl.BlockSpec((512,), lambda i: (0,)),
            pl.BlockSpec((512, C), lambda i: (0, 0)),
            pl.BlockSpec((C,), lambda i: (0,)),
        ],
        out_specs=[
            pl.BlockSpec((TILE_R, C), lambda i: (i, 0)),
            pl.BlockSpec((TILE_R, 2 * N_CAM), lambda i: (i, 0)),
        ],
        out_shape=[
            jax.ShapeDtypeStruct((R_TOT, C), jnp.float32),
            jax.ShapeDtypeStruct((R_TOT, 2 * N_CAM), jnp.float32),
        ],
    )(rp_flat, pts4, l2it, fm, W1, b1, W2, b2)

    sf = out.reshape(1, 64, 64, 4, C).transpose(0, 4, 3, 1, 2)
    spc = uv.reshape(4096, 4, N_CAM, 2).transpose(2, 0, 1, 3)
    spc = spc.reshape(1, N_CAM, 4096, 1, 4, 2)
    return sf, spc
